# baseline (speedup 1.0000x reference)
import functools

import jax
import jax.numpy as jnp
from jax import lax
from jax.experimental import pallas as pl
from jax.experimental.pallas import tpu as pltpu
from jax.experimental.pallas import tpu_sc as plsc

_B = 16384
_D = 32
_V = 1_000_000
_NC = 2
_NS = 16
_NW = _NC * _NS
_BPW = _B // _NW
_L = 16

_NP = (_V + 127) // 128
_PPT = (_NP + _NW - 1) // _NW
_VP = _NP * 128


def _detile_body(t2_hbm, out_hbm, panel_v, rows_v):
    wid = lax.axis_index("s") * _NC + lax.axis_index("c")
    p0 = wid * _PPT
    n = jnp.minimum(_PPT, _NP - p0)
    lane = lax.iota(jnp.int32, _L)

    def do_panel(k, carry):
        p = p0 + k
        pltpu.sync_copy(t2_hbm.at[:, pl.ds(p * 128, 128)], panel_v)

        def do_row(r, carry2):
            rv = jnp.full((_L,), r, jnp.int32)
            a = plsc.load_gather(panel_v, [lane, rv])
            b = plsc.load_gather(panel_v, [lane + _L, rv])
            rows_v[pl.ds(r * _D, _L)] = a
            rows_v[pl.ds(r * _D + _L, _L)] = b
            return carry2

        lax.fori_loop(0, 128, do_row, 0)
        pltpu.sync_copy(rows_v, out_hbm.at[pl.ds(p * 128 * _D, 128 * _D)])
        return carry

    lax.fori_loop(0, n, do_panel, 0)


_detile = functools.partial(
    pl.kernel,
    mesh=plsc.VectorSubcoreMesh(core_axis_name="c", subcore_axis_name="s"),
    out_type=jax.ShapeDtypeStruct((_VP * _D,), jnp.float32),
    compiler_params=pltpu.CompilerParams(
        needs_layout_passes=False, use_tc_tiling_on_sc=True
    ),
    scratch_types=[
        pltpu.VMEM((_D, 128), jnp.float32),
        pltpu.VMEM((128 * _D,), jnp.float32),
    ],
)(_detile_body)


def _tec_body(rows_hbm, c1_hbm, c2_hbm, out_hbm,
              idx1_v, idx2_v, rows1_v, rows2_v, out_v, sem1, sem2):
    wid = lax.axis_index("s") * _NC + lax.axis_index("c")
    base = wid * _BPW

    pltpu.sync_copy(c1_hbm.at[pl.ds(base, _BPW)], idx1_v)
    pltpu.sync_copy(c2_hbm.at[pl.ds(base, _BPW)], idx2_v)
    g1 = pltpu.async_copy(rows_hbm.at[idx1_v], rows1_v, sem1)
    g2 = pltpu.async_copy(rows_hbm.at[idx2_v], rows2_v, sem2)
    g1.wait()
    g2.wait()

    lane = lax.iota(jnp.int32, _L)

    def blk_body(blk, carry):
        row_idx = blk * _L + lane
        acc = jnp.zeros((_L,), jnp.float32)
        for j in range(_D):
            jv = jnp.full((_L,), j, jnp.int32)
            a = plsc.load_gather(rows1_v, [row_idx, jv])
            b = plsc.load_gather(rows2_v, [row_idx, jv])
            acc = acc + a * b
        out_v[pl.ds(blk * _L, _L)] = acc
        return carry

    lax.fori_loop(0, _BPW // _L, blk_body, 0)

    pltpu.sync_copy(out_v, out_hbm.at[pl.ds(base, _BPW)])


_gather_dot = functools.partial(
    pl.kernel,
    mesh=plsc.VectorSubcoreMesh(core_axis_name="c", subcore_axis_name="s"),
    out_type=jax.ShapeDtypeStruct((_B,), jnp.float32),
    compiler_params=pltpu.CompilerParams(
        needs_layout_passes=False, use_tc_tiling_on_sc=False
    ),
    scratch_types=[
        pltpu.VMEM((_BPW,), jnp.int32),
        pltpu.VMEM((_BPW,), jnp.int32),
        pltpu.VMEM((_BPW, _D), jnp.float32),
        pltpu.VMEM((_BPW, _D), jnp.float32),
        pltpu.VMEM((_BPW,), jnp.float32),
        pltpu.SemaphoreType.DMA,
        pltpu.SemaphoreType.DMA,
    ],
)(_tec_body)


@jax.jit
def kernel(champ1, champ2, table):
    c1 = champ1.astype(jnp.int32)
    c2 = champ2.astype(jnp.int32)
    t2 = jnp.swapaxes(table, 0, 1)
    rows = _detile(t2).reshape(_VP, _D)
    dot = _gather_dot(rows, c1, c2)
    return dot.reshape(-1, 1, 1)


# speedup vs baseline: 1.4706x; 1.4706x over previous
import functools

import jax
import jax.numpy as jnp
from jax import lax
from jax.experimental import pallas as pl
from jax.experimental.pallas import tpu as pltpu
from jax.experimental.pallas import tpu_sc as plsc

_B = 16384
_D = 32
_V = 1_000_000
_NC = 2
_NS = 16
_NW = _NC * _NS
_BPW = _B // _NW
_L = 16

_NP = (_V + 127) // 128
_PPT = (_NP + _NW - 1) // _NW
_VP = _NP * 128


def _detile_body(t2_hbm, out_hbm, panel_v, rows_v,
                 sem_in0, sem_in1, sem_out0, sem_out1):
    wid = lax.axis_index("s") * _NC + lax.axis_index("c")
    p0 = wid * _PPT
    n = jnp.minimum(_PPT, _NP - p0)
    lane = lax.iota(jnp.int32, _L)

    def in_copy(k, slot, sem):
        return pltpu.make_async_copy(
            t2_hbm.at[:, pl.ds((p0 + k) * 128, 128)], panel_v.at[slot], sem
        )

    def out_copy(k, slot, sem):
        return pltpu.make_async_copy(
            rows_v.at[pl.ds(slot * 128 * _D, 128 * _D)],
            out_hbm.at[pl.ds((p0 + k) * 128 * _D, 128 * _D)],
            sem,
        )

    in_copy(0, 0, sem_in0).start()

    def _even(k):
        slot = 0

        @pl.when(k >= 2)
        def _():
            out_copy(k - 2, slot, sem_out0).wait()

        in_copy(k, slot, sem_in0).wait()

        @pl.when(k + 1 < n)
        def _():
            in_copy(k + 1, 1 - slot, sem_in1).start()

    def _odd(k):
        slot = 1

        @pl.when(k >= 2)
        def _():
            out_copy(k - 2, slot, sem_out1).wait()

        in_copy(k, slot, sem_in1).wait()

        @pl.when(k + 1 < n)
        def _():
            in_copy(k + 1, 1 - slot, sem_in0).start()

    def do_panel(k, carry):
        slot = k & 1
        lax.cond(slot == 0, _even, _odd, k)

        pan = panel_v.at[slot]
        dst = rows_v.at[pl.ds(slot * 128 * _D, 128 * _D)]
        for rb in range(8):
            r0 = rb * _L
            rr = r0 + lane
            pos = rr * _D
            for j in range(_D):
                v = plsc.load_gather(pan, [jnp.full((_L,), j, jnp.int32), rr])
                plsc.store_scatter(dst, [pos + j], v)

        lax.cond(slot == 0,
                 lambda k: out_copy(k, 0, sem_out0).start(),
                 lambda k: out_copy(k, 1, sem_out1).start(), k)
        return carry

    lax.fori_loop(0, n, do_panel, 0)
    lax.cond((n & 1) == 0,
             lambda: out_copy(n - 2, 0, sem_out0).wait(),
             lambda: out_copy(n - 2, 1, sem_out1).wait())
    lax.cond((n & 1) == 0,
             lambda: out_copy(n - 1, 1, sem_out1).wait(),
             lambda: out_copy(n - 1, 0, sem_out0).wait())


_detile = functools.partial(
    pl.kernel,
    mesh=plsc.VectorSubcoreMesh(core_axis_name="c", subcore_axis_name="s"),
    out_type=jax.ShapeDtypeStruct((_VP * _D,), jnp.float32),
    compiler_params=pltpu.CompilerParams(
        needs_layout_passes=False, use_tc_tiling_on_sc=True
    ),
    scratch_types=[
        pltpu.VMEM((2, _D, 128), jnp.float32),
        pltpu.VMEM((2 * 128 * _D,), jnp.float32),
        pltpu.SemaphoreType.DMA,
        pltpu.SemaphoreType.DMA,
        pltpu.SemaphoreType.DMA,
        pltpu.SemaphoreType.DMA,
    ],
)(_detile_body)


def _tec_body(rows_hbm, c1_hbm, c2_hbm, out_hbm,
              idx1_v, idx2_v, rows1_v, rows2_v, out_v, sem1, sem2):
    wid = lax.axis_index("s") * _NC + lax.axis_index("c")
    base = wid * _BPW

    pltpu.sync_copy(c1_hbm.at[pl.ds(base, _BPW)], idx1_v)
    pltpu.sync_copy(c2_hbm.at[pl.ds(base, _BPW)], idx2_v)
    g1 = pltpu.async_copy(rows_hbm.at[idx1_v], rows1_v, sem1)
    g2 = pltpu.async_copy(rows_hbm.at[idx2_v], rows2_v, sem2)
    g1.wait()
    g2.wait()

    lane = lax.iota(jnp.int32, _L)

    def blk_body(blk, carry):
        row_idx = blk * _L + lane
        acc = jnp.zeros((_L,), jnp.float32)
        for j in range(_D):
            jv = jnp.full((_L,), j, jnp.int32)
            a = plsc.load_gather(rows1_v, [row_idx, jv])
            b = plsc.load_gather(rows2_v, [row_idx, jv])
            acc = acc + a * b
        out_v[pl.ds(blk * _L, _L)] = acc
        return carry

    lax.fori_loop(0, _BPW // _L, blk_body, 0)

    pltpu.sync_copy(out_v, out_hbm.at[pl.ds(base, _BPW)])


_gather_dot = functools.partial(
    pl.kernel,
    mesh=plsc.VectorSubcoreMesh(core_axis_name="c", subcore_axis_name="s"),
    out_type=jax.ShapeDtypeStruct((_B,), jnp.float32),
    compiler_params=pltpu.CompilerParams(
        needs_layout_passes=False, use_tc_tiling_on_sc=False
    ),
    scratch_types=[
        pltpu.VMEM((_BPW,), jnp.int32),
        pltpu.VMEM((_BPW,), jnp.int32),
        pltpu.VMEM((_BPW, _D), jnp.float32),
        pltpu.VMEM((_BPW, _D), jnp.float32),
        pltpu.VMEM((_BPW,), jnp.float32),
        pltpu.SemaphoreType.DMA,
        pltpu.SemaphoreType.DMA,
    ],
)(_tec_body)


@jax.jit
def kernel(champ1, champ2, table):
    c1 = champ1.astype(jnp.int32)
    c2 = champ2.astype(jnp.int32)
    t2 = jnp.swapaxes(table, 0, 1)
    rows = _detile(t2).reshape(_VP, _D)
    dot = _gather_dot(rows, c1, c2)
    return dot.reshape(-1, 1, 1)


# speedup vs baseline: 2.1036x; 1.4304x over previous
import functools

import jax
import jax.numpy as jnp
from jax import lax
from jax.experimental import pallas as pl
from jax.experimental.pallas import tpu as pltpu
from jax.experimental.pallas import tpu_sc as plsc

_B = 16384
_D = 32
_V = 1_000_000
_NC = 2
_NS = 16
_NW = _NC * _NS
_BPW = _B // _NW
_L = 16

_CB = 2048
_NBLK = (_V + _CB - 1) // _CB
_OUTR = _V * _D // 128


def _detile_body(in_ref, out_ref, scr):
    scr[...] = in_ref[...].T
    for q in range(4):
        out_ref[:, q * 32:(q + 1) * 32] = scr[pl.Slice(q, _CB // 4, 4), :]


_detile = pl.pallas_call(
    _detile_body,
    grid=(_NBLK,),
    in_specs=[pl.BlockSpec((_D, _CB), lambda i: (0, i))],
    out_specs=pl.BlockSpec((_CB * _D // 128, 128), lambda i: (i, 0)),
    out_shape=jax.ShapeDtypeStruct((_OUTR, 128), jnp.float32),
    scratch_shapes=[pltpu.VMEM((_CB, _D), jnp.float32)],
)


def _tec_body(rows_hbm, c1_hbm, c2_hbm, out_hbm,
              idx1_v, idx2_v, rows1_v, rows2_v, out_v, sem1, sem2):
    wid = lax.axis_index("s") * _NC + lax.axis_index("c")
    base = wid * _BPW

    pltpu.sync_copy(c1_hbm.at[pl.ds(base, _BPW)], idx1_v)
    pltpu.sync_copy(c2_hbm.at[pl.ds(base, _BPW)], idx2_v)
    g1 = pltpu.async_copy(rows_hbm.at[idx1_v], rows1_v, sem1)
    g2 = pltpu.async_copy(rows_hbm.at[idx2_v], rows2_v, sem2)
    g1.wait()
    g2.wait()

    lane = lax.iota(jnp.int32, _L)

    def blk_body(blk, carry):
        row_idx = blk * _L + lane
        acc = jnp.zeros((_L,), jnp.float32)
        for j in range(_D):
            jv = jnp.full((_L,), j, jnp.int32)
            a = plsc.load_gather(rows1_v, [row_idx, jv])
            b = plsc.load_gather(rows2_v, [row_idx, jv])
            acc = acc + a * b
        out_v[pl.ds(blk * _L, _L)] = acc
        return carry

    lax.fori_loop(0, _BPW // _L, blk_body, 0)

    pltpu.sync_copy(out_v, out_hbm.at[pl.ds(base, _BPW)])


_gather_dot = functools.partial(
    pl.kernel,
    mesh=plsc.VectorSubcoreMesh(core_axis_name="c", subcore_axis_name="s"),
    out_type=jax.ShapeDtypeStruct((_B,), jnp.float32),
    compiler_params=pltpu.CompilerParams(
        needs_layout_passes=False, use_tc_tiling_on_sc=False
    ),
    scratch_types=[
        pltpu.VMEM((_BPW,), jnp.int32),
        pltpu.VMEM((_BPW,), jnp.int32),
        pltpu.VMEM((_BPW, _D), jnp.float32),
        pltpu.VMEM((_BPW, _D), jnp.float32),
        pltpu.VMEM((_BPW,), jnp.float32),
        pltpu.SemaphoreType.DMA,
        pltpu.SemaphoreType.DMA,
    ],
)(_tec_body)


@jax.jit
def kernel(champ1, champ2, table):
    c1 = champ1.astype(jnp.int32)
    c2 = champ2.astype(jnp.int32)
    t2 = jnp.swapaxes(table, 0, 1)
    rows = _detile(t2).reshape(_V, _D)
    dot = _gather_dot(rows, c1, c2)
    return dot.reshape(-1, 1, 1)
